# baseline (speedup 1.0000x reference)
import functools

import jax
import jax.numpy as jnp
from jax import lax
from jax.experimental import pallas as pl
from jax.experimental.pallas import tpu as pltpu
from jax.experimental.pallas import tpu_sc as plsc

N = 10000
E = 320000
D = 128
P = 4096
NC = 2
NS = 16
NW = NC * NS
EC = E // NW
CP = 128
ECP = 10112
NCHUNK = ECP // CP
NP = 10240
RT = NP // NS
ZR = 128
PC = P // NW

_mesh = plsc.VectorSubcoreMesh(core_axis_name="c", subcore_axis_name="s")
_sc_params = pltpu.CompilerParams(needs_layout_passes=False)
_f32 = jnp.float32


@functools.partial(
    pl.kernel,
    out_type=jax.ShapeDtypeStruct((2 * NW * N,), _f32),
    mesh=_mesh,
    compiler_params=_sc_params,
    scratch_types=[
        pltpu.VMEM((EC,), jnp.int32),
        pltpu.VMEM((EC,), jnp.int32),
        pltpu.VMEM((N,), _f32),
        pltpu.VMEM((N,), _f32),
    ],
)
def _sc_degrees(src_hbm, dst_hbm, out_hbm, src_v, dst_v, hsrc_v, hdst_v):
    cid = lax.axis_index("c")
    sid = lax.axis_index("s")
    wid = sid * NC + cid
    pltpu.sync_copy(src_hbm.at[pl.ds(wid * EC, EC)], src_v)
    pltpu.sync_copy(dst_hbm.at[pl.ds(wid * EC, EC)], dst_v)
    zeros16 = jnp.zeros((16,), _f32)
    ones16 = jnp.ones((16,), _f32)

    def zero(i, _):
        hsrc_v[pl.ds(i * 16, 16)] = zeros16
        hdst_v[pl.ds(i * 16, 16)] = zeros16
        return 0

    lax.fori_loop(0, N // 16, zero, 0)

    def count(i, _):
        plsc.addupdate_scatter(hsrc_v, [src_v[pl.ds(i * 16, 16)]], ones16)
        plsc.addupdate_scatter(hdst_v, [dst_v[pl.ds(i * 16, 16)]], ones16)
        return 0

    lax.fori_loop(0, EC // 16, count, 0)
    pltpu.sync_copy(hsrc_v, out_hbm.at[pl.ds(wid * N, N)])
    pltpu.sync_copy(hdst_v, out_hbm.at[pl.ds((NW + wid) * N, N)])


@functools.partial(
    pl.kernel,
    out_type=jax.ShapeDtypeStruct((2, NP, D), _f32),
    mesh=_mesh,
    compiler_params=_sc_params,
    scratch_types=[
        pltpu.VMEM((2, 3, CP), jnp.int32),
        pltpu.VMEM((2, CP, D), _f32),
        pltpu.VMEM_SHARED((NP, D), _f32),
        pltpu.SemaphoreType.DMA,
        pltpu.SemaphoreType.DMA,
        pltpu.SemaphoreType.DMA,
        pltpu.SemaphoreType.DMA,
    ],
)
def _sc_agg(h_hbm, ed_hbm, out_hbm, ebuf_v, rows_v, acc,
            esem0, esem1, gsem0, gsem1):
    cid = lax.axis_index("c")
    sid = lax.axis_index("s")
    wid = sid * NC + cid
    esem = (esem0, esem1)
    gsem = (gsem0, gsem1)

    def stage_e(i, b):
        pltpu.async_copy(ed_hbm.at[wid, i], ebuf_v.at[b], esem[b])

    def wait_e(i, b):
        pltpu.make_async_copy(ed_hbm.at[wid, i], ebuf_v.at[b],
                              esem[b]).wait()

    def issue_g(b):
        pltpu.async_copy(h_hbm.at[ebuf_v.at[b, 0]], rows_v.at[b], gsem[b])

    def wait_g(b):
        pltpu.make_async_copy(h_hbm.at[ebuf_v.at[b, 0]], rows_v.at[b],
                              gsem[b]).wait()

    def mul_chunk(b):
        def body(e, _):
            wv = plsc.bitcast(
                plsc.load_gather(
                    ebuf_v.at[b], [jnp.full((16,), 2, jnp.int32),
                                   jnp.full((16,), e, jnp.int32)]), _f32)
            for j in range(D // 16):
                sl = pl.ds(j * 16, 16)
                rows_v[b, e, sl] = rows_v[b, e, sl] * wv
            return 0

        lax.fori_loop(0, CP, body, 0, unroll=4)

    def scatter(b):
        pltpu.sync_copy(rows_v.at[b], acc.at[ebuf_v.at[b, 1]], add=True)

    zeros16 = jnp.zeros((16,), _f32)

    def zrow(r, _):
        for j in range(D // 16):
            rows_v[0, r, pl.ds(j * 16, 16)] = zeros16
        return 0

    lax.fori_loop(0, CP, zrow, 0)
    for k in range(RT // CP):
        pltpu.sync_copy(rows_v.at[0], acc.at[pl.ds(sid * RT + k * CP, CP)])

    stage_e(0, 0)
    stage_e(1, 1)
    wait_e(0, 0)
    issue_g(0)
    plsc.subcore_barrier()

    def step(i, b, do_next, do_stage):
        wait_g(b)
        if do_next:
            wait_e(i + 1, 1 - b)
            issue_g(1 - b)
        mul_chunk(b)
        scatter(b)
        if do_stage:
            stage_e(i + 2, b)

    def pair(t, _):
        step(2 * t, 0, True, True)
        step(2 * t + 1, 1, True, True)
        return 0

    lax.fori_loop(0, (NCHUNK - 3) // 2, pair, 0)
    step(NCHUNK - 3, 0, True, True)
    step(NCHUNK - 2, 1, True, False)
    step(NCHUNK - 1, 0, False, False)

    plsc.subcore_barrier()
    pltpu.sync_copy(acc.at[pl.ds(sid * RT, RT)],
                    out_hbm.at[cid, pl.ds(sid * RT, RT)])


@functools.partial(
    pl.kernel,
    out_type=jax.ShapeDtypeStruct((2, P, D), _f32),
    mesh=_mesh,
    compiler_params=_sc_params,
    scratch_types=[
        pltpu.VMEM((PC,), jnp.int32),
        pltpu.VMEM((PC,), jnp.int32),
        pltpu.VMEM((PC, D), _f32),
        pltpu.VMEM((PC, D), _f32),
        pltpu.SemaphoreType.DMA,
    ],
)
def _sc_pairs(h_hbm, pos_hbm, neg_hbm, out_hbm, a_v, b_v, ra_v, rb_v, sem):
    cid = lax.axis_index("c")
    sid = lax.axis_index("s")
    wid = sid * NC + cid
    for g, g_hbm in enumerate((pos_hbm, neg_hbm)):
        pltpu.sync_copy(g_hbm.at[0, pl.ds(wid * PC, PC)], a_v)
        pltpu.sync_copy(g_hbm.at[1, pl.ds(wid * PC, PC)], b_v)
        pltpu.async_copy(h_hbm.at[a_v], ra_v, sem).wait()
        pltpu.async_copy(h_hbm.at[b_v], rb_v, sem).wait()

        def mul(e, _):
            for j in range(D // 16):
                sl = pl.ds(j * 16, 16)
                ra_v[e, sl] = ra_v[e, sl] * rb_v[e, sl]
            return 0

        lax.fori_loop(0, PC, mul, 0)
        pltpu.sync_copy(ra_v, out_hbm.at[g, pl.ds(wid * PC, PC)])


_R = 400


def _tc_disq_body(degp_ref, disq_ref):
    deg = jnp.sum(degp_ref[...], axis=1)
    disq_ref[...] = lax.rsqrt(jnp.maximum(deg, 1.0))


def _tc_disq(degp):
    return pl.pallas_call(
        _tc_disq_body,
        out_shape=jax.ShapeDtypeStruct((2, N), _f32),
    )(degp)


def _tc_scale_body(x_ref, disq_ref, h0_ref):
    h0_ref[...] = x_ref[...] * disq_ref[:, 0:1]


def _tc_scale(x, disq_t):
    return pl.pallas_call(
        _tc_scale_body,
        grid=(N // _R,),
        in_specs=[
            pl.BlockSpec((_R, D), lambda i: (i, 0)),
            pl.BlockSpec((_R, 2), lambda i: (i, 0)),
        ],
        out_specs=pl.BlockSpec((_R, D), lambda i: (i, 0)),
        out_shape=jax.ShapeDtypeStruct((N, D), _f32),
    )(x, disq_t)


def _tc_dense_body(p_ref, disq_ref, w_ref, b_ref, flag_ref, out_ref):
    agg = (p_ref[0] + p_ref[1]) * disq_ref[:, 1:2]
    h = jnp.dot(agg, w_ref[...], preferred_element_type=_f32,
                precision=lax.Precision.HIGHEST) + b_ref[...]
    f = flag_ref[0, 0]
    out_ref[...] = f * h + (1.0 - f) * (jnp.maximum(h, 0.0) * disq_ref[:, 0:1])


def _tc_dense(p2, disq_t, w, b, flag):
    return pl.pallas_call(
        _tc_dense_body,
        grid=(N // _R,),
        in_specs=[
            pl.BlockSpec((2, _R, D), lambda i: (0, i, 0)),
            pl.BlockSpec((_R, 2), lambda i: (i, 0)),
            pl.BlockSpec((D, D), lambda i: (0, 0)),
            pl.BlockSpec((1, D), lambda i: (0, 0)),
            pl.BlockSpec((1, 1), lambda i: (0, 0)),
        ],
        out_specs=pl.BlockSpec((_R, D), lambda i: (i, 0)),
        out_shape=jax.ShapeDtypeStruct((N, D), _f32),
    )(p2, disq_t, w, b.reshape(1, D), flag.reshape(1, 1))


_RM = 1024


def _tc_mlp_body(e_ref, w0_ref, b0_ref, w1_ref, b1_ref, w2_ref, b2_ref,
                 out_ref):
    h = jnp.dot(e_ref[...], w0_ref[...], preferred_element_type=_f32,
                precision=lax.Precision.HIGHEST) + b0_ref[...]
    h = jnp.maximum(h, 0.0)
    h = jnp.dot(h, w1_ref[...], preferred_element_type=_f32,
                precision=lax.Precision.HIGHEST) + b1_ref[...]
    h = jnp.maximum(h, 0.0)
    out_ref[...] = jnp.dot(h, w2_ref[...], preferred_element_type=_f32,
                           precision=lax.Precision.HIGHEST) + b2_ref[...]


def _tc_mlp(e2, pw0, pb0, pw1, pb1, pw2, pb2):
    return pl.pallas_call(
        _tc_mlp_body,
        grid=(2 * P // _RM,),
        in_specs=[
            pl.BlockSpec((_RM, D), lambda i: (i, 0)),
            pl.BlockSpec((D, D), lambda i: (0, 0)),
            pl.BlockSpec((1, D), lambda i: (0, 0)),
            pl.BlockSpec((D, D), lambda i: (0, 0)),
            pl.BlockSpec((1, D), lambda i: (0, 0)),
            pl.BlockSpec((D, 1), lambda i: (0, 0)),
            pl.BlockSpec((1, 1), lambda i: (0, 0)),
        ],
        out_specs=pl.BlockSpec((_RM, 1), lambda i: (i, 0)),
        out_shape=jax.ShapeDtypeStruct((2 * P, 1), _f32),
    )(e2, pw0, pb0.reshape(1, D), pw1, pb1.reshape(1, D), pw2,
      pb2.reshape(1, 1))


def kernel(x, edge_weight, W0, b0, W1, b1, W2, b2, PW0, Pb0, PW1, Pb1,
           PW2, Pb2, edge_index, pos_edge_index, neg_edge_index):
    src = edge_index[0]
    dst = edge_index[1]

    degp = _sc_degrees(src, dst).reshape(2, NW, N)
    disq_t = _tc_disq(degp).T
    h0 = _tc_scale(x, disq_t)

    pad = ((0, 0), (0, ECP - EC))
    src3 = jnp.pad(src.reshape(NW, EC), pad).reshape(1, NW, NCHUNK, 1, CP)
    dst3 = jnp.pad(dst.reshape(NW, EC), pad).reshape(1, NW, NCHUNK, 1, CP)
    ew3 = lax.bitcast_convert_type(
        jnp.pad(edge_weight.reshape(3, NW, EC), ((0, 0),) + pad),
        jnp.int32).reshape(3, NW, NCHUNK, 1, CP)
    eds = jnp.concatenate(
        (jnp.broadcast_to(src3, (3, NW, NCHUNK, 1, CP)),
         jnp.broadcast_to(dst3, (3, NW, NCHUNK, 1, CP)), ew3), axis=3)
    ws = jnp.stack((W0, W1, W2))
    bs = jnp.stack((b0, b1, b2))
    flags = jnp.array([0.0, 0.0, 1.0], _f32)

    def layer(h, xs):
        ed, w, b, flag = xs
        p2 = _sc_agg(h, ed)
        return _tc_dense(p2, disq_t, w, b, flag), None

    h, _ = lax.scan(layer, h0, (eds, ws, bs, flags))
    prod = _sc_pairs(h, pos_edge_index, neg_edge_index)
    out = _tc_mlp(prod.reshape(2 * P, D), PW0, Pb0, PW1, Pb1, PW2, Pb2)
    return out[:P], out[P:]


# speedup vs baseline: 1.6570x; 1.6570x over previous
import functools

import jax
import jax.numpy as jnp
from jax import lax
from jax.experimental import pallas as pl
from jax.experimental.pallas import tpu as pltpu
from jax.experimental.pallas import tpu_sc as plsc

N = 10000
E = 320000
D = 128
P = 4096
NC = 2
NS = 16
NW = NC * NS
EC = E // NW
C = 80
NCHUNK = EC // C
NP = 10240
RT = NP // NS
ZR = 128
PC = P // NW

_mesh = plsc.VectorSubcoreMesh(core_axis_name="c", subcore_axis_name="s")
_sc_params = pltpu.CompilerParams(needs_layout_passes=False)
_f32 = jnp.float32


@functools.partial(
    pl.kernel,
    out_type=jax.ShapeDtypeStruct((2 * NW * N,), _f32),
    mesh=_mesh,
    compiler_params=_sc_params,
    scratch_types=[
        pltpu.VMEM((EC,), jnp.int32),
        pltpu.VMEM((EC,), jnp.int32),
        pltpu.VMEM((N,), _f32),
        pltpu.VMEM((N,), _f32),
    ],
)
def _sc_degrees(src_hbm, dst_hbm, out_hbm, src_v, dst_v, hsrc_v, hdst_v):
    cid = lax.axis_index("c")
    sid = lax.axis_index("s")
    wid = sid * NC + cid
    pltpu.sync_copy(src_hbm.at[pl.ds(wid * EC, EC)], src_v)
    pltpu.sync_copy(dst_hbm.at[pl.ds(wid * EC, EC)], dst_v)
    zeros16 = jnp.zeros((16,), _f32)
    ones16 = jnp.ones((16,), _f32)

    def zero(i, _):
        hsrc_v[pl.ds(i * 16, 16)] = zeros16
        hdst_v[pl.ds(i * 16, 16)] = zeros16
        return 0

    lax.fori_loop(0, N // 16, zero, 0)

    def count(i, _):
        plsc.addupdate_scatter(hsrc_v, [src_v[pl.ds(i * 16, 16)]], ones16)
        plsc.addupdate_scatter(hdst_v, [dst_v[pl.ds(i * 16, 16)]], ones16)
        return 0

    lax.fori_loop(0, EC // 16, count, 0)
    pltpu.sync_copy(hsrc_v, out_hbm.at[pl.ds(wid * N, N)])
    pltpu.sync_copy(hdst_v, out_hbm.at[pl.ds((NW + wid) * N, N)])


@functools.partial(
    pl.kernel,
    out_type=jax.ShapeDtypeStruct((2, NP, D), _f32),
    mesh=_mesh,
    compiler_params=_sc_params,
    scratch_types=[
        pltpu.VMEM((3, 3, C), jnp.int32),
        pltpu.VMEM((2, C, D), _f32),
        pltpu.VMEM_SHARED((NP, D), _f32),
        pltpu.SemaphoreType.DMA,
        pltpu.SemaphoreType.DMA,
        pltpu.SemaphoreType.DMA,
        pltpu.SemaphoreType.DMA,
        pltpu.SemaphoreType.DMA,
        pltpu.SemaphoreType.DMA,
        pltpu.SemaphoreType.DMA,
    ],
)
def _sc_agg(h_hbm, ed_hbm, out_hbm, ebuf_v, rows_v, acc,
            esem0, esem1, esem2, gsem0, gsem1, ssem0, ssem1):
    cid = lax.axis_index("c")
    sid = lax.axis_index("s")
    wid = sid * NC + cid
    esem = (esem0, esem1, esem2)
    gsem = (gsem0, gsem1)
    ssem = (ssem0, ssem1)

    def stage_e(i, eb):
        pltpu.async_copy(ed_hbm.at[wid, i], ebuf_v.at[eb], esem[eb])

    def wait_e(i, eb):
        pltpu.make_async_copy(ed_hbm.at[wid, i], ebuf_v.at[eb],
                              esem[eb]).wait()

    def issue_g(eb, rb):
        pltpu.async_copy(h_hbm.at[ebuf_v.at[eb, 0]], rows_v.at[rb], gsem[rb])

    def wait_g(eb, rb):
        pltpu.make_async_copy(h_hbm.at[ebuf_v.at[eb, 0]], rows_v.at[rb],
                              gsem[rb]).wait()

    def issue_s(eb, rb):
        pltpu.async_copy(rows_v.at[rb], acc.at[ebuf_v.at[eb, 1]], ssem[rb],
                         add=True)

    def wait_s(eb, rb):
        pltpu.make_async_copy(rows_v.at[rb], acc.at[ebuf_v.at[eb, 1]],
                              ssem[rb]).wait()

    def mul_chunk(eb, rb):
        def body(e, _):
            wv = plsc.bitcast(
                plsc.load_gather(
                    ebuf_v.at[eb], [jnp.full((16,), 2, jnp.int32),
                                    jnp.full((16,), e, jnp.int32)]), _f32)
            for j in range(D // 16):
                sl = pl.ds(j * 16, 16)
                rows_v[rb, e, sl] = rows_v[rb, e, sl] * wv
            return 0

        lax.fori_loop(0, C, body, 0, unroll=4)

    zeros16 = jnp.zeros((16,), _f32)

    def zrow(r, _):
        for j in range(D // 16):
            rows_v[0, r, pl.ds(j * 16, 16)] = zeros16
        return 0

    lax.fori_loop(0, C, zrow, 0)
    for k in range(RT // C):
        pltpu.sync_copy(rows_v.at[0], acc.at[pl.ds(sid * RT + k * C, C)])

    def step(i, first, last):
        rb = i % 2
        eb = i % 3
        wait_g(eb, rb)
        if not first:
            wait_s((i - 1) % 3, 1 - rb)
        if i + 2 < NCHUNK:
            stage_e(i + 2, (i + 2) % 3)
        if i + 1 < NCHUNK:
            wait_e(i + 1, (i + 1) % 3)
            issue_g((i + 1) % 3, 1 - rb)
        mul_chunk(eb, rb)
        issue_s(eb, rb)
        if last:
            wait_s(eb, rb)

    stage_e(0, 0)
    stage_e(1, 1)
    wait_e(0, 0)
    issue_g(0, 0)
    plsc.subcore_barrier()

    step(0, True, False)
    step(1, False, False)

    def six(t, _):
        for k in range(6):
            step6(2 + 6 * t + k, k)
        return 0

    def step6(i, k):
        rb = k % 2
        eb = (2 + k) % 3
        wait_g(eb, rb)
        wait_s((eb + 2) % 3, 1 - rb)
        stage_e(i + 2, (eb + 2) % 3)
        wait_e(i + 1, (eb + 1) % 3)
        issue_g((eb + 1) % 3, 1 - rb)
        mul_chunk(eb, rb)
        issue_s(eb, rb)

    lax.fori_loop(0, (NCHUNK - 5) // 6, six, 0)
    step(NCHUNK - 3, False, False)
    step(NCHUNK - 2, False, False)
    step(NCHUNK - 1, False, True)

    plsc.subcore_barrier()
    pltpu.sync_copy(acc.at[pl.ds(sid * RT, RT)],
                    out_hbm.at[cid, pl.ds(sid * RT, RT)])


@functools.partial(
    pl.kernel,
    out_type=jax.ShapeDtypeStruct((2, P, D), _f32),
    mesh=_mesh,
    compiler_params=_sc_params,
    scratch_types=[
        pltpu.VMEM((PC,), jnp.int32),
        pltpu.VMEM((PC,), jnp.int32),
        pltpu.VMEM((PC, D), _f32),
        pltpu.VMEM((PC, D), _f32),
        pltpu.SemaphoreType.DMA,
    ],
)
def _sc_pairs(h_hbm, pos_hbm, neg_hbm, out_hbm, a_v, b_v, ra_v, rb_v, sem):
    cid = lax.axis_index("c")
    sid = lax.axis_index("s")
    wid = sid * NC + cid
    for g, g_hbm in enumerate((pos_hbm, neg_hbm)):
        pltpu.sync_copy(g_hbm.at[0, pl.ds(wid * PC, PC)], a_v)
        pltpu.sync_copy(g_hbm.at[1, pl.ds(wid * PC, PC)], b_v)
        pltpu.async_copy(h_hbm.at[a_v], ra_v, sem).wait()
        pltpu.async_copy(h_hbm.at[b_v], rb_v, sem).wait()

        def mul(e, _):
            for j in range(D // 16):
                sl = pl.ds(j * 16, 16)
                ra_v[e, sl] = ra_v[e, sl] * rb_v[e, sl]
            return 0

        lax.fori_loop(0, PC, mul, 0)
        pltpu.sync_copy(ra_v, out_hbm.at[g, pl.ds(wid * PC, PC)])


_R = 400


def _tc_disq_body(degp_ref, disq_ref):
    deg = jnp.sum(degp_ref[...], axis=1)
    disq_ref[...] = lax.rsqrt(jnp.maximum(deg, 1.0))


def _tc_disq(degp):
    return pl.pallas_call(
        _tc_disq_body,
        out_shape=jax.ShapeDtypeStruct((2, N), _f32),
    )(degp)


def _tc_scale_body(x_ref, disq_ref, h0_ref):
    h0_ref[...] = x_ref[...] * disq_ref[:, 0:1]


def _tc_scale(x, disq_t):
    return pl.pallas_call(
        _tc_scale_body,
        grid=(N // _R,),
        in_specs=[
            pl.BlockSpec((_R, D), lambda i: (i, 0)),
            pl.BlockSpec((_R, 2), lambda i: (i, 0)),
        ],
        out_specs=pl.BlockSpec((_R, D), lambda i: (i, 0)),
        out_shape=jax.ShapeDtypeStruct((N, D), _f32),
    )(x, disq_t)


def _tc_dense_body(p_ref, disq_ref, w_ref, b_ref, flag_ref, out_ref):
    agg = (p_ref[0] + p_ref[1]) * disq_ref[:, 1:2]
    h = jnp.dot(agg, w_ref[...], preferred_element_type=_f32,
                precision=lax.Precision.HIGHEST) + b_ref[...]
    f = flag_ref[0, 0]
    out_ref[...] = f * h + (1.0 - f) * (jnp.maximum(h, 0.0) * disq_ref[:, 0:1])


def _tc_dense(p2, disq_t, w, b, flag):
    return pl.pallas_call(
        _tc_dense_body,
        grid=(N // _R,),
        in_specs=[
            pl.BlockSpec((2, _R, D), lambda i: (0, i, 0)),
            pl.BlockSpec((_R, 2), lambda i: (i, 0)),
            pl.BlockSpec((D, D), lambda i: (0, 0)),
            pl.BlockSpec((1, D), lambda i: (0, 0)),
            pl.BlockSpec((1, 1), lambda i: (0, 0)),
        ],
        out_specs=pl.BlockSpec((_R, D), lambda i: (i, 0)),
        out_shape=jax.ShapeDtypeStruct((N, D), _f32),
    )(p2, disq_t, w, b.reshape(1, D), flag.reshape(1, 1))


_RM = 1024


def _tc_mlp_body(e_ref, w0_ref, b0_ref, w1_ref, b1_ref, w2_ref, b2_ref,
                 out_ref):
    h = jnp.dot(e_ref[...], w0_ref[...], preferred_element_type=_f32,
                precision=lax.Precision.HIGHEST) + b0_ref[...]
    h = jnp.maximum(h, 0.0)
    h = jnp.dot(h, w1_ref[...], preferred_element_type=_f32,
                precision=lax.Precision.HIGHEST) + b1_ref[...]
    h = jnp.maximum(h, 0.0)
    out_ref[...] = jnp.dot(h, w2_ref[...], preferred_element_type=_f32,
                           precision=lax.Precision.HIGHEST) + b2_ref[...]


def _tc_mlp(e2, pw0, pb0, pw1, pb1, pw2, pb2):
    return pl.pallas_call(
        _tc_mlp_body,
        grid=(2 * P // _RM,),
        in_specs=[
            pl.BlockSpec((_RM, D), lambda i: (i, 0)),
            pl.BlockSpec((D, D), lambda i: (0, 0)),
            pl.BlockSpec((1, D), lambda i: (0, 0)),
            pl.BlockSpec((D, D), lambda i: (0, 0)),
            pl.BlockSpec((1, D), lambda i: (0, 0)),
            pl.BlockSpec((D, 1), lambda i: (0, 0)),
            pl.BlockSpec((1, 1), lambda i: (0, 0)),
        ],
        out_specs=pl.BlockSpec((_RM, 1), lambda i: (i, 0)),
        out_shape=jax.ShapeDtypeStruct((2 * P, 1), _f32),
    )(e2, pw0, pb0.reshape(1, D), pw1, pb1.reshape(1, D), pw2,
      pb2.reshape(1, 1))


def kernel(x, edge_weight, W0, b0, W1, b1, W2, b2, PW0, Pb0, PW1, Pb1,
           PW2, Pb2, edge_index, pos_edge_index, neg_edge_index):
    src = edge_index[0]
    dst = edge_index[1]

    degp = _sc_degrees(src, dst).reshape(2, NW, N)
    disq_t = _tc_disq(degp).T
    h0 = _tc_scale(x, disq_t)

    src3 = src.reshape(1, NW, NCHUNK, 1, C)
    dst3 = dst.reshape(1, NW, NCHUNK, 1, C)
    ew3 = lax.bitcast_convert_type(edge_weight, jnp.int32).reshape(
        3, NW, NCHUNK, 1, C)
    eds = jnp.concatenate(
        (jnp.broadcast_to(src3, (3, NW, NCHUNK, 1, C)),
         jnp.broadcast_to(dst3, (3, NW, NCHUNK, 1, C)), ew3), axis=3)
    ws = jnp.stack((W0, W1, W2))
    bs = jnp.stack((b0, b1, b2))
    flags = jnp.array([0.0, 0.0, 1.0], _f32)

    def layer(h, xs):
        ed, w, b, flag = xs
        p2 = _sc_agg(h, ed)
        return _tc_dense(p2, disq_t, w, b, flag), None

    h, _ = lax.scan(layer, h0, (eds, ws, bs, flags))
    prod = _sc_pairs(h, pos_edge_index, neg_edge_index)
    out = _tc_mlp(prod.reshape(2 * P, D), PW0, Pb0, PW1, Pb1, PW2, Pb2)
    return out[:P], out[P:]
